# baseline (speedup 1.0000x reference)
import jax
import jax.numpy as jnp
from jax.experimental import pallas as pl
from jax.experimental.pallas import tpu as pltpu
from functools import partial


def _egat_body(H_ref, E_ref, Ehi_ref, Elo_ref, AH0_ref, AE0_ref, ME_ref,
               WHt_ref, bH_ref, WEt_ref, bE_ref, avec_ref, a3c_ref,
               Hp_ref, Ep_ref, Hm_ref, Ht_s, t1_s, t2_s,
               *, blk_i, sub, n, m, d):
    ib = pl.program_id(1)

    @pl.when(ib == 0)
    def _per_batch_init():
        Ht0 = jnp.dot(H_ref[0], WHt_ref[...],
                      preferred_element_type=jnp.float32) + bH_ref[...]
        Ht_s[...] = Ht0
        a1 = avec_ref[0:1, :]
        a2 = avec_ref[1:2, :]
        t1_s[...] = jax.lax.dot_general(Ht0, a1, (((1,), (1,)), ((), ())),
                                        preferred_element_type=jnp.float32)
        t2_s[...] = jax.lax.dot_general(a2, Ht0, (((1,), (1,)), ((), ())),
                                        preferred_element_type=jnp.float32)
        Etr = jnp.dot(E_ref[0], WEt_ref[...],
                      preferred_element_type=jnp.float32) + bE_ref[...]
        wEm = jnp.where(AE0_ref[...] > 0.0, 1.0 / m, 0.0).astype(jnp.float32)
        Ep_ref[0] = jnp.dot(wEm, Etr, preferred_element_type=jnp.float32)

    Ht = Ht_s[...]
    t2 = t2_s[...]
    MEv = ME_ref[0]

    MEhi = MEv.astype(jnp.bfloat16)
    MElo = (MEv - MEhi.astype(jnp.float32)).astype(jnp.bfloat16)
    Ehi = Ehi_ref[0]
    Elo = Elo_ref[0]
    Estar = (jnp.dot(MEhi, Ehi, preferred_element_type=jnp.float32)
             + jnp.dot(MEhi, Elo, preferred_element_type=jnp.float32)
             + jnp.dot(MElo, Ehi, preferred_element_type=jnp.float32))

    Eshi = Estar.astype(jnp.bfloat16)
    Eslo = (Estar - Eshi.astype(jnp.float32)).astype(jnp.bfloat16)
    s2 = jnp.dot(Eshi, a3c_ref[...], preferred_element_type=jnp.float32)
    s2b = jnp.dot(Eslo, a3c_ref[...], preferred_element_type=jnp.float32)
    s3 = s2[:, 0:1] + s2[:, 1:2] + s2b[:, 0:1]
    t1b = t1_s[pl.ds(ib * blk_i, blk_i), :]
    raw = t1b + t2 + s3.reshape(blk_i, n)
    raw = jnp.where(raw >= 0, raw, 0.2 * raw)
    mask = AH0_ref[...] > 0.0
    scores = jnp.where(mask, raw, -9e15)
    mx = jnp.max(scores, axis=-1, keepdims=True)
    e = jnp.exp(scores - mx)
    w = e / jnp.sum(e, axis=-1, keepdims=True)
    wm = w * mask.astype(jnp.float32)

    Hp_ref[0] = jnp.dot(wm, Ht, preferred_element_type=jnp.float32)
    prod = (Estar.reshape(blk_i, n, d) * wm[:, :, None]) * Ht[None, :, :]
    Hm_ref[0] = jnp.sum(prod, axis=1)


def kernel(H, E, AH, AE, ME, MH, W_H_w, W_H_b, W_E_w, W_E_b, a, b,
           blk_i: int = 64, sub: int = 64, interpret: bool = False):
    Bs, Nn, Dd = H.shape
    Mm = E.shape[1]
    nb = Nn // blk_i
    AH0 = AH[0]
    AE0 = AE[0]
    avec = a[:, 0].reshape(3, Dd)
    Ehi = E.astype(jnp.bfloat16)
    Elo = (E - Ehi.astype(jnp.float32)).astype(jnp.bfloat16)
    a3col = a[2 * Dd:, :]
    a3hi = a3col.astype(jnp.bfloat16)
    a3lo = (a3col - a3hi.astype(jnp.float32)).astype(jnp.bfloat16)
    a3c = jnp.concatenate([a3hi, a3lo], axis=1)
    WHt = W_H_w.T
    WEt = W_E_w.T
    bH = W_H_b.reshape(1, Dd)
    bE = W_E_b.reshape(1, Dd)

    grid = (Bs, nb)
    out_shapes = (
        jax.ShapeDtypeStruct((Bs, Nn, Dd), jnp.float32),
        jax.ShapeDtypeStruct((Bs, Mm, Dd), jnp.float32),
        jax.ShapeDtypeStruct((Bs, Nn, Dd), jnp.float32),
    )
    in_specs = [
        pl.BlockSpec((1, Nn, Dd), lambda bb, i: (bb, 0, 0)),
        pl.BlockSpec((1, Mm, Dd), lambda bb, i: (bb, 0, 0)),
        pl.BlockSpec((1, Mm, Dd), lambda bb, i: (bb, 0, 0)),
        pl.BlockSpec((1, Mm, Dd), lambda bb, i: (bb, 0, 0)),
        pl.BlockSpec((blk_i, Nn), lambda bb, i: (i, 0)),
        pl.BlockSpec((Mm, Mm), lambda bb, i: (0, 0)),
        pl.BlockSpec((1, blk_i * Nn, Mm), lambda bb, i: (bb, i, 0)),
        pl.BlockSpec((Dd, Dd), lambda bb, i: (0, 0)),
        pl.BlockSpec((1, Dd), lambda bb, i: (0, 0)),
        pl.BlockSpec((Dd, Dd), lambda bb, i: (0, 0)),
        pl.BlockSpec((1, Dd), lambda bb, i: (0, 0)),
        pl.BlockSpec((3, Dd), lambda bb, i: (0, 0)),
        pl.BlockSpec((Dd, 2), lambda bb, i: (0, 0)),
    ]
    out_specs = (
        pl.BlockSpec((1, blk_i, Dd), lambda bb, i: (bb, i, 0)),
        pl.BlockSpec((1, Mm, Dd), lambda bb, i: (bb, 0, 0)),
        pl.BlockSpec((1, blk_i, Dd), lambda bb, i: (bb, i, 0)),
    )
    f = pl.pallas_call(
        partial(_egat_body, blk_i=blk_i, sub=sub, n=Nn, m=Mm, d=Dd),
        grid=grid,
        in_specs=in_specs,
        out_specs=out_specs,
        out_shape=out_shapes,
        scratch_shapes=[pltpu.VMEM((Nn, Dd), jnp.float32),
                        pltpu.VMEM((Nn, 1), jnp.float32),
                        pltpu.VMEM((1, Nn), jnp.float32)],
        compiler_params=pltpu.CompilerParams(
            dimension_semantics=("parallel", "arbitrary"),
            vmem_limit_bytes=100 * 1024 * 1024,
        ),
        interpret=interpret,
    )
    Hp, Ep, Hm = f(H, E, Ehi, Elo, AH0, AE0, ME, WHt, bH, WEt, bE, avec, a3c)
    return Hp, Ep, Hm


# speedup vs baseline: 2.3810x; 2.3810x over previous
import jax
import jax.numpy as jnp
from jax.experimental import pallas as pl
from jax.experimental.pallas import tpu as pltpu
from functools import partial


def _egat_body(H_ref, E_ref, AH0_ref, AE0_ref, ME_ref, WHt_ref, bH_ref,
               WEt_ref, bE_ref, avec_ref,
               Hp_ref, Ep_ref, Hm_ref, Ht_s, t1_s, t2_s,
               *, blk_i, n, m, d):
    ib = pl.program_id(1)

    @pl.when(ib == 0)
    def _per_batch_init():
        Ht0 = jnp.dot(H_ref[0], WHt_ref[...],
                      preferred_element_type=jnp.float32) + bH_ref[...]
        Ht_s[...] = Ht0
        a1 = avec_ref[0:1, :]
        a2 = avec_ref[1:2, :]
        t1_s[...] = jax.lax.dot_general(Ht0, a1, (((1,), (1,)), ((), ())),
                                        preferred_element_type=jnp.float32)
        t2_s[...] = jax.lax.dot_general(a2, Ht0, (((1,), (1,)), ((), ())),
                                        preferred_element_type=jnp.float32)
        Etr = jnp.dot(E_ref[0], WEt_ref[...],
                      preferred_element_type=jnp.float32) + bE_ref[...]
        wEm = jnp.where(AE0_ref[...] > 0.0, 1.0 / m, 0.0).astype(jnp.float32)
        Ep_ref[0] = jnp.dot(wEm, Etr, preferred_element_type=jnp.float32)

    Ht = Ht_s[...]
    t2 = t2_s[...]

    Estar = jnp.dot(ME_ref[0], E_ref[0],
                    preferred_element_type=jnp.float32)
    a3 = avec_ref[2:3, :]
    s3 = jax.lax.dot_general(Estar, a3, (((1,), (1,)), ((), ())),
                             preferred_element_type=jnp.float32)
    t1b = t1_s[pl.ds(ib * blk_i, blk_i), :]
    raw = t1b + t2 + s3.reshape(blk_i, n)
    raw = jnp.where(raw >= 0, raw, 0.2 * raw)
    mask = AH0_ref[...] > 0.0
    scores = jnp.where(mask, raw, -9e15)
    mx = jnp.max(scores, axis=-1, keepdims=True)
    e = jnp.exp(scores - mx)
    w = e / jnp.sum(e, axis=-1, keepdims=True)
    wm = w * mask.astype(jnp.float32)

    Hp_ref[0] = jnp.dot(wm, Ht, preferred_element_type=jnp.float32)
    prod = (Estar.reshape(blk_i, n, d) * wm[:, :, None]) * Ht[None, :, :]
    Hm_ref[0] = jnp.sum(prod, axis=1)


def kernel(H, E, AH, AE, ME, MH, W_H_w, W_H_b, W_E_w, W_E_b, a, b,
           blk_i: int = 64, interpret: bool = False):
    Bs, Nn, Dd = H.shape
    Mm = E.shape[1]
    nb = Nn // blk_i
    AH0 = AH[0]
    AE0 = AE[0]
    avec = a[:, 0].reshape(3, Dd)
    WHt = W_H_w.T
    WEt = W_E_w.T
    bH = W_H_b.reshape(1, Dd)
    bE = W_E_b.reshape(1, Dd)

    grid = (Bs, nb)
    out_shapes = (
        jax.ShapeDtypeStruct((Bs, Nn, Dd), jnp.float32),
        jax.ShapeDtypeStruct((Bs, Mm, Dd), jnp.float32),
        jax.ShapeDtypeStruct((Bs, Nn, Dd), jnp.float32),
    )
    in_specs = [
        pl.BlockSpec((1, Nn, Dd), lambda bb, i: (bb, 0, 0)),
        pl.BlockSpec((1, Mm, Dd), lambda bb, i: (bb, 0, 0)),
        pl.BlockSpec((blk_i, Nn), lambda bb, i: (i, 0)),
        pl.BlockSpec((Mm, Mm), lambda bb, i: (0, 0)),
        pl.BlockSpec((1, blk_i * Nn, Mm), lambda bb, i: (bb, i, 0)),
        pl.BlockSpec((Dd, Dd), lambda bb, i: (0, 0)),
        pl.BlockSpec((1, Dd), lambda bb, i: (0, 0)),
        pl.BlockSpec((Dd, Dd), lambda bb, i: (0, 0)),
        pl.BlockSpec((1, Dd), lambda bb, i: (0, 0)),
        pl.BlockSpec((3, Dd), lambda bb, i: (0, 0)),
    ]
    out_specs = (
        pl.BlockSpec((1, blk_i, Dd), lambda bb, i: (bb, i, 0)),
        pl.BlockSpec((1, Mm, Dd), lambda bb, i: (bb, 0, 0)),
        pl.BlockSpec((1, blk_i, Dd), lambda bb, i: (bb, i, 0)),
    )
    f = pl.pallas_call(
        partial(_egat_body, blk_i=blk_i, n=Nn, m=Mm, d=Dd),
        grid=grid,
        in_specs=in_specs,
        out_specs=out_specs,
        out_shape=out_shapes,
        scratch_shapes=[pltpu.VMEM((Nn, Dd), jnp.float32),
                        pltpu.VMEM((Nn, 1), jnp.float32),
                        pltpu.VMEM((1, Nn), jnp.float32)],
        compiler_params=pltpu.CompilerParams(
            dimension_semantics=("parallel", "arbitrary"),
            vmem_limit_bytes=100 * 1024 * 1024,
        ),
        interpret=interpret,
    )
    Hp, Ep, Hm = f(H, E, AH0, AE0, ME, WHt, bH, WEt, bE, avec)
    return Hp, Ep, Hm
